# baseline (speedup 1.0000x reference)
import functools

import jax
import jax.numpy as jnp
from jax import lax
from jax.experimental import pallas as pl
from jax.experimental.pallas import tpu as pltpu
from jax.experimental.pallas import tpu_sc as plsc

D_MODEL = 1024
LANES = 16
NUM_CORES = 2
NUM_SUBCORES = 16
NUM_WORKERS = NUM_CORES * NUM_SUBCORES
CHUNK = 16
NBUF = 3
AHEAD = 2


@functools.partial(jax.jit, static_argnames=("total_tok",))
def _pe_add(flat, idx2d, pe, total_tok):
    tok_per_w = total_tok // NUM_WORKERS
    num_chunks = tok_per_w // CHUNK
    mesh = plsc.VectorSubcoreMesh(
        core_axis_name="c", subcore_axis_name="s",
        num_cores=NUM_CORES, num_subcores=NUM_SUBCORES,
    )

    @functools.partial(
        pl.kernel,
        out_type=jax.ShapeDtypeStruct((total_tok, D_MODEL), jnp.float32),
        mesh=mesh,
        scratch_types=[
            pltpu.VMEM((num_chunks, CHUNK), jnp.int32),
            pltpu.VMEM((NBUF, CHUNK, D_MODEL), jnp.float32),
            pltpu.VMEM((NBUF, CHUNK, D_MODEL), jnp.float32),
            pltpu.SemaphoreType.DMA((NBUF,)),
            pltpu.SemaphoreType.DMA((NBUF,)),
            pltpu.SemaphoreType.DMA((NBUF,)),
        ],
    )
    def body(flat_hbm, idx_hbm, pe_hbm, out_hbm,
             idx_v, rows_v, flat_v, gsem, fsem, osem):
        wid = lax.axis_index("s") * NUM_CORES + lax.axis_index("c")
        base = wid * tok_per_w

        pltpu.sync_copy(idx_hbm.at[pl.ds(wid * num_chunks, num_chunks)], idx_v)

        def start(i, b):
            off = base + i * CHUNK
            pltpu.async_copy(pe_hbm.at[idx_v.at[i]], rows_v.at[b], gsem.at[b])
            pltpu.async_copy(flat_hbm.at[pl.ds(off, CHUNK)], flat_v.at[b],
                             fsem.at[b])

        def drain_store(i, b):
            off = base + i * CHUNK
            pltpu.make_async_copy(flat_v.at[b], out_hbm.at[pl.ds(off, CHUNK)],
                                  osem.at[b]).wait()

        def finish(i, b):
            off = base + i * CHUNK
            pltpu.make_async_copy(pe_hbm.at[idx_v.at[i]], rows_v.at[b],
                                  gsem.at[b]).wait()
            pltpu.make_async_copy(flat_hbm.at[pl.ds(off, CHUNK)],
                                  flat_v.at[b], fsem.at[b]).wait()

            @pl.loop(0, CHUNK)
            def _row(r):
                for c in range(D_MODEL // LANES):
                    s = pl.ds(c * LANES, LANES)
                    plsc.addupdate(flat_v.at[b, r, s], rows_v[b, r, s])

            pltpu.async_copy(flat_v.at[b], out_hbm.at[pl.ds(off, CHUNK)],
                             osem.at[b])

        for j in range(AHEAD):
            start(j, j)

        @pl.loop(0, num_chunks, step=NBUF)
        def _grp(g):
            for b in range(NBUF):
                i = g + b

                @pl.when(i < num_chunks)
                def _():
                    finish(i, b)
                    nxt = i + AHEAD

                    @pl.when(nxt < num_chunks)
                    def _():
                        @pl.when(nxt - NBUF >= 0)
                        def _():
                            drain_store(nxt - NBUF, (nxt % NBUF))

                        start(nxt, (nxt % NBUF))

        for t in range(num_chunks - (NBUF - AHEAD), num_chunks):
            drain_store(t, t % NBUF)

    return body(flat, idx2d, pe)


def kernel(flat, positions, cu_seqlens, pe):
    del cu_seqlens
    total_tok = flat.shape[0]
    idx2d = positions.astype(jnp.int32).reshape(total_tok // CHUNK, CHUNK)
    return _pe_add(flat, idx2d, pe, total_tok)


# speedup vs baseline: 1.0971x; 1.0971x over previous
import functools

import jax
import jax.numpy as jnp
from jax import lax
from jax.experimental import pallas as pl
from jax.experimental.pallas import tpu as pltpu
from jax.experimental.pallas import tpu_sc as plsc

D_MODEL = 1024
LANES = 16
NUM_CORES = 2
NUM_SUBCORES = 16
NUM_WORKERS = NUM_CORES * NUM_SUBCORES
CHUNK = 8
NBUF = 6
AHEAD = 4


@functools.partial(jax.jit, static_argnames=("total_tok",))
def _pe_add(flat, idx2d, pe, total_tok):
    tok_per_w = total_tok // NUM_WORKERS
    num_chunks = tok_per_w // CHUNK
    mesh = plsc.VectorSubcoreMesh(
        core_axis_name="c", subcore_axis_name="s",
        num_cores=NUM_CORES, num_subcores=NUM_SUBCORES,
    )

    @functools.partial(
        pl.kernel,
        out_type=jax.ShapeDtypeStruct((total_tok, D_MODEL), jnp.float32),
        mesh=mesh,
        scratch_types=[
            pltpu.VMEM((num_chunks, CHUNK), jnp.int32),
            pltpu.VMEM((NBUF, CHUNK, D_MODEL), jnp.float32),
            pltpu.VMEM((NBUF, CHUNK, D_MODEL), jnp.float32),
            pltpu.SemaphoreType.DMA((NBUF,)),
            pltpu.SemaphoreType.DMA((NBUF,)),
            pltpu.SemaphoreType.DMA((NBUF,)),
        ],
    )
    def body(flat_hbm, idx_hbm, pe_hbm, out_hbm,
             idx_v, rows_v, flat_v, gsem, fsem, osem):
        wid = lax.axis_index("s") * NUM_CORES + lax.axis_index("c")
        base = wid * tok_per_w

        pltpu.sync_copy(idx_hbm.at[pl.ds(wid * num_chunks, num_chunks)], idx_v)

        def start(i, b):
            off = base + i * CHUNK
            pltpu.async_copy(pe_hbm.at[idx_v.at[i]], rows_v.at[b], gsem.at[b])
            pltpu.async_copy(flat_hbm.at[pl.ds(off, CHUNK)], flat_v.at[b],
                             fsem.at[b])

        def drain_store(i, b):
            off = base + i * CHUNK
            pltpu.make_async_copy(flat_v.at[b], out_hbm.at[pl.ds(off, CHUNK)],
                                  osem.at[b]).wait()

        def finish(i, b):
            off = base + i * CHUNK
            pltpu.make_async_copy(pe_hbm.at[idx_v.at[i]], rows_v.at[b],
                                  gsem.at[b]).wait()
            pltpu.make_async_copy(flat_hbm.at[pl.ds(off, CHUNK)],
                                  flat_v.at[b], fsem.at[b]).wait()

            @pl.loop(0, CHUNK)
            def _row(r):
                for c in range(D_MODEL // LANES):
                    s = pl.ds(c * LANES, LANES)
                    plsc.addupdate(flat_v.at[b, r, s], rows_v[b, r, s])

            pltpu.async_copy(flat_v.at[b], out_hbm.at[pl.ds(off, CHUNK)],
                             osem.at[b])

        for j in range(AHEAD):
            start(j, j)

        @pl.loop(0, num_chunks, step=NBUF)
        def _grp(g):
            for b in range(NBUF):
                i = g + b

                @pl.when(i < num_chunks)
                def _():
                    finish(i, b)
                    nxt = i + AHEAD

                    @pl.when(nxt < num_chunks)
                    def _():
                        @pl.when(nxt - NBUF >= 0)
                        def _():
                            drain_store(nxt - NBUF, (nxt % NBUF))

                        start(nxt, (nxt % NBUF))

        for t in range(num_chunks - (NBUF - AHEAD), num_chunks):
            drain_store(t, t % NBUF)

    return body(flat, idx2d, pe)


def kernel(flat, positions, cu_seqlens, pe):
    del cu_seqlens
    total_tok = flat.shape[0]
    idx2d = positions.astype(jnp.int32).reshape(total_tok // CHUNK, CHUNK)
    return _pe_add(flat, idx2d, pe, total_tok)
